# Initial kernel scaffold; baseline (speedup 1.0000x reference)
import jax
import jax.numpy as jnp
from jax.experimental import pallas as pl


def kernel(weeks, minutes, global_spatial_idx, edge_index, traffic_h, local_batch_idx, local_spatial_idx, local_spatial_feature, params):
    raise NotImplementedError("write your pallas kernel here")


# speedup vs baseline: 3.1686x; 3.1686x over previous
import functools
import math

import jax
import jax.numpy as jnp
from jax import lax
from jax.experimental import pallas as pl
from jax.experimental.pallas import tpu as pltpu
from jax.experimental.pallas import tpu_sc as plsc

N = 10000
E = 320000
NP = 10240
GB = 64
LINKS = 100000
S = 12
NTILES = 32
EPT = E // NTILES
K = 200
NCH = EPT // K
STRIPE = NP // 16
EPS = 1e-5

_mesh = lambda: plsc.VectorSubcoreMesh(core_axis_name="c", subcore_axis_name="s")


@functools.partial(jax.jit, static_argnames=("F",))
def _sc_agg(h, src, dst, zeros_stripe, F):
    @functools.partial(
        pl.kernel,
        out_type=jax.ShapeDtypeStruct((2 * NP, F), jnp.float32),
        mesh=_mesh(),
        scratch_types=[
            pltpu.VMEM((K,), jnp.int32),
            pltpu.VMEM((K,), jnp.int32),
            pltpu.VMEM((K, F), jnp.float32),
            pltpu.VMEM_SHARED((NP, F), jnp.float32),
            pltpu.SemaphoreType.DMA,
        ],
    )
    def k(h_hbm, src_hbm, dst_hbm, z_hbm, out_hbm, src_v, dst_v, rows_v, acc, sem):
        c = lax.axis_index("c")
        s = lax.axis_index("s")
        pltpu.sync_copy(z_hbm, acc.at[pl.ds(s * STRIPE, STRIPE)])
        plsc.subcore_barrier()
        base = (c * 16 + s) * EPT

        def body(i, carry):
            off = base + i * K
            pltpu.sync_copy(src_hbm.at[pl.ds(off, K)], src_v)
            pltpu.sync_copy(dst_hbm.at[pl.ds(off, K)], dst_v)
            pltpu.async_copy(h_hbm.at[src_v], rows_v, sem).wait()
            pltpu.sync_copy(rows_v, acc.at[dst_v], add=True)
            return carry

        lax.fori_loop(0, NCH, body, 0)
        plsc.subcore_barrier()
        pltpu.sync_copy(
            acc.at[pl.ds(s * STRIPE, STRIPE)],
            out_hbm.at[pl.ds(c * NP + s * STRIPE, STRIPE)],
        )

    out = k(h, src, dst, zeros_stripe)
    return out.reshape(2, NP, F)


@jax.jit
def _sc_degrees(src, dst, ones_rows, zeros_stripe):
    @functools.partial(
        pl.kernel,
        out_type=(
            jax.ShapeDtypeStruct((2 * NP, 128), jnp.float32),
            jax.ShapeDtypeStruct((2 * NP, 128), jnp.float32),
        ),
        mesh=_mesh(),
        scratch_types=[
            pltpu.VMEM((K,), jnp.int32),
            pltpu.VMEM((K, 128), jnp.float32),
            pltpu.VMEM_SHARED((NP, 128), jnp.float32),
        ],
    )
    def k(src_hbm, dst_hbm, ones_hbm, z_hbm, oo_hbm, oi_hbm,
          idx_v, ones_v, acc):
        c = lax.axis_index("c")
        s = lax.axis_index("s")
        sl = pl.ds(s * STRIPE, STRIPE)
        ol = pl.ds(c * NP + s * STRIPE, STRIPE)
        base = (c * 16 + s) * EPT
        pltpu.sync_copy(ones_hbm, ones_v)
        pltpu.sync_copy(z_hbm, acc.at[sl])
        plsc.subcore_barrier()

        def pass_over(idx_hbm, out_hbm, needs_rezero):
            def body(i, carry):
                pltpu.sync_copy(idx_hbm.at[pl.ds(base + i * K, K)], idx_v)
                pltpu.sync_copy(ones_v, acc.at[idx_v], add=True)
                return carry

            lax.fori_loop(0, NCH, body, 0)
            plsc.subcore_barrier()
            pltpu.sync_copy(acc.at[sl], out_hbm.at[ol])
            if needs_rezero:
                pltpu.sync_copy(z_hbm, acc.at[sl])
                plsc.subcore_barrier()

        pass_over(src_hbm, oo_hbm, True)
        pass_over(dst_hbm, oi_hbm, False)

    oo, oi = k(src, dst, ones_rows, zeros_stripe)
    return oo.reshape(2, NP, 128), oi.reshape(2, NP, 128)


@functools.partial(jax.jit, static_argnames=("F",))
def _sc_gather(table, idx, F):
    BPT = NP // NTILES

    @functools.partial(
        pl.kernel,
        out_type=jax.ShapeDtypeStruct((NP, F), jnp.float32),
        mesh=_mesh(),
        scratch_types=[
            pltpu.VMEM((BPT,), jnp.int32),
            pltpu.VMEM((BPT, F), jnp.float32),
            pltpu.SemaphoreType.DMA,
        ],
    )
    def k(table_hbm, idx_hbm, out_hbm, idx_v, rows_v, sem):
        c = lax.axis_index("c")
        s = lax.axis_index("s")
        base = (c * 16 + s) * BPT
        pltpu.sync_copy(idx_hbm.at[pl.ds(base, BPT)], idx_v)
        pltpu.async_copy(table_hbm.at[idx_v], rows_v, sem).wait()
        pltpu.sync_copy(rows_v, out_hbm.at[pl.ds(base, BPT)])

    return k(table, idx)


def _gelu(x):
    return 0.5 * x * (1.0 + lax.erf(x * (1.0 / math.sqrt(2.0))))


def _bn_stats(x, rb):
    R, C = x.shape

    def body(x_ref, s1_ref, s2_ref):
        i = pl.program_id(0)
        xb = x_ref[...]
        ps = jnp.sum(xb, axis=0, keepdims=True)
        pss = jnp.sum(xb * xb, axis=0, keepdims=True)

        @pl.when(i == 0)
        def _():
            s1_ref[...] = ps
            s2_ref[...] = pss

        @pl.when(i > 0)
        def _():
            s1_ref[...] += ps
            s2_ref[...] += pss

    return pl.pallas_call(
        body,
        grid=(R // rb,),
        in_specs=[pl.BlockSpec((rb, C), lambda i: (i, 0))],
        out_specs=(pl.BlockSpec((1, C), lambda i: (0, 0)),
                   pl.BlockSpec((1, C), lambda i: (0, 0))),
        out_shape=(jax.ShapeDtypeStruct((1, C), jnp.float32),
                   jax.ShapeDtypeStruct((1, C), jnp.float32)),
    )(x)


def _bn_apply(x, s1, s2, rb, scale=None):
    R, C = x.shape
    n = float(R)
    with_scale = scale is not None

    def body(*refs):
        if with_scale:
            x_ref, s1_ref, s2_ref, sc_ref, o_ref = refs
        else:
            x_ref, s1_ref, s2_ref, o_ref = refs
        m = s1_ref[...] / n
        v = s2_ref[...] / n - m * m
        inv = lax.rsqrt(v + EPS)
        y = (x_ref[...] - m) * inv
        if with_scale:
            y = y * sc_ref[...]
        o_ref[...] = y

    in_specs = [pl.BlockSpec((rb, C), lambda i: (i, 0)),
                pl.BlockSpec((1, C), lambda i: (0, 0)),
                pl.BlockSpec((1, C), lambda i: (0, 0))]
    args = [x, s1, s2]
    if with_scale:
        in_specs.append(pl.BlockSpec((rb, 1), lambda i: (i, 0)))
        args.append(scale)
    return pl.pallas_call(
        body,
        grid=(R // rb,),
        in_specs=in_specs,
        out_specs=pl.BlockSpec((rb, C), lambda i: (i, 0)),
        out_shape=jax.ShapeDtypeStruct((R, C), jnp.float32),
    )(*args)


def _mm(x0, W, b, rb, x1=None, pre=None, post=None, res=None, act=True):
    R, Kd = x0.shape
    M = W.shape[1]
    f_x1, f_pre, f_post, f_res = (x1 is not None, pre is not None,
                                  post is not None, res is not None)

    def body(*refs):
        it = iter(refs)
        x_ref = next(it)
        x1_ref = next(it) if f_x1 else None
        W_ref = next(it)
        b_ref = next(it)
        pre_ref = next(it) if f_pre else None
        post_ref = next(it) if f_post else None
        res_ref = next(it) if f_res else None
        o_ref = next(it)
        x = x_ref[...]
        if f_x1:
            x = x + x1_ref[...]
        if f_pre:
            x = x * pre_ref[...]
        y = jnp.dot(x, W_ref[...], preferred_element_type=jnp.float32) + b_ref[...]
        if act:
            y = _gelu(y)
        if f_post:
            y = y * post_ref[...]
        if f_res:
            y = y + res_ref[...]
        o_ref[...] = y

    in_specs = [pl.BlockSpec((rb, Kd), lambda i: (i, 0))]
    args = [x0]
    if f_x1:
        in_specs.append(pl.BlockSpec((rb, Kd), lambda i: (i, 0)))
        args.append(x1)
    in_specs += [pl.BlockSpec((Kd, M), lambda i: (0, 0)),
                 pl.BlockSpec((1, M), lambda i: (0, 0))]
    args += [W, b.reshape(1, M)]
    if f_pre:
        in_specs.append(pl.BlockSpec((rb, 1), lambda i: (i, 0)))
        args.append(pre)
    if f_post:
        in_specs.append(pl.BlockSpec((rb, 1), lambda i: (i, 0)))
        args.append(post)
    if f_res:
        in_specs.append(pl.BlockSpec((rb, M), lambda i: (i, 0)))
        args.append(res)
    return pl.pallas_call(
        body,
        grid=(R // rb,),
        in_specs=in_specs,
        out_specs=pl.BlockSpec((rb, M), lambda i: (i, 0)),
        out_shape=jax.ShapeDtypeStruct((R, M), jnp.float32),
    )(*args)


def _pick_half(rows, parity, rb):
    R = rows.shape[0]

    def body(r_ref, p_ref, o_ref):
        o_ref[...] = jnp.where(p_ref[...] > 0, r_ref[:, 64:128], r_ref[:, 0:64])

    return pl.pallas_call(
        body,
        grid=(R // rb,),
        in_specs=[pl.BlockSpec((rb, 128), lambda i: (i, 0)),
                  pl.BlockSpec((rb, 1), lambda i: (i, 0))],
        out_specs=pl.BlockSpec((rb, 64), lambda i: (i, 0)),
        out_shape=jax.ShapeDtypeStruct((R, 64), jnp.float32),
    )(rows, parity)


def _deg_scale(a0, a1):
    R = a0.shape[0]
    rb = 1000

    def body(a_ref, b_ref, o_ref):
        o_ref[...] = lax.rsqrt(jnp.maximum(a_ref[...] + b_ref[...], 1.0))

    return pl.pallas_call(
        body,
        grid=(R // rb,),
        in_specs=[pl.BlockSpec((rb, 1), lambda i: (i, 0)),
                  pl.BlockSpec((rb, 1), lambda i: (i, 0))],
        out_specs=pl.BlockSpec((rb, 1), lambda i: (i, 0)),
        out_shape=jax.ShapeDtypeStruct((R, 1), jnp.float32),
    )(a0, a1)


def _rowscale(x, sc, rb):
    R, C = x.shape

    def body(x_ref, s_ref, o_ref):
        o_ref[...] = x_ref[...] * s_ref[...]

    return pl.pallas_call(
        body,
        grid=(R // rb,),
        in_specs=[pl.BlockSpec((rb, C), lambda i: (i, 0)),
                  pl.BlockSpec((rb, 1), lambda i: (i, 0))],
        out_specs=pl.BlockSpec((rb, C), lambda i: (i, 0)),
        out_shape=jax.ShapeDtypeStruct((R, C), jnp.float32),
    )(x, sc)


def _lstm(x, WihT, WhhT, bias, H, rb, want_seq, want_mean, res=None):
    B, S_, I = x.shape
    f_res = res is not None

    def body(*refs):
        it = iter(refs)
        x_ref = next(it)
        wi_ref = next(it)
        wh_ref = next(it)
        b_ref = next(it)
        res_ref = next(it) if f_res else None
        outs = [next(it) for _ in range(int(want_seq) + int(want_mean))]
        h = jnp.zeros((rb, H), jnp.float32)
        c = jnp.zeros((rb, H), jnp.float32)
        acc = jnp.zeros((rb, H), jnp.float32)
        wi = wi_ref[...]
        wh = wh_ref[...]
        bb = b_ref[...]
        for t in range(S_):
            xt = x_ref[:, t, :]
            g = (jnp.dot(xt, wi, preferred_element_type=jnp.float32)
                 + jnp.dot(h, wh, preferred_element_type=jnp.float32) + bb)
            i_g = jax.nn.sigmoid(g[:, 0:H])
            f_g = jax.nn.sigmoid(g[:, H:2 * H])
            g_g = jnp.tanh(g[:, 2 * H:3 * H])
            o_g = jax.nn.sigmoid(g[:, 3 * H:4 * H])
            c = f_g * c + i_g * g_g
            h = o_g * jnp.tanh(c)
            oi = 0
            if want_seq:
                y = h
                if f_res:
                    y = y + res_ref[:, t, :]
                outs[oi][:, t, :] = y
                oi += 1
            if want_mean:
                acc = acc + h
        if want_mean:
            outs[-1][...] = acc * (1.0 / S_)

    in_specs = [pl.BlockSpec((rb, S_, I), lambda i: (i, 0, 0)),
                pl.BlockSpec((I, 4 * H), lambda i: (0, 0)),
                pl.BlockSpec((H, 4 * H), lambda i: (0, 0)),
                pl.BlockSpec((1, 4 * H), lambda i: (0, 0))]
    args = [x, WihT, WhhT, bias.reshape(1, -1)]
    if f_res:
        in_specs.append(pl.BlockSpec((rb, S_, H), lambda i: (i, 0, 0)))
        args.append(res)
    out_specs, out_shape = [], []
    if want_seq:
        out_specs.append(pl.BlockSpec((rb, S_, H), lambda i: (i, 0, 0)))
        out_shape.append(jax.ShapeDtypeStruct((B, S_, H), jnp.float32))
    if want_mean:
        out_specs.append(pl.BlockSpec((rb, H), lambda i: (i, 0)))
        out_shape.append(jax.ShapeDtypeStruct((B, H), jnp.float32))
    outs = pl.pallas_call(
        body,
        grid=(B // rb,),
        in_specs=in_specs,
        out_specs=tuple(out_specs),
        out_shape=tuple(out_shape),
    )(*args)
    return outs


def _seg_mean(x, lbi, rb):
    N_, F = x.shape
    fb = min(F, 512)
    nblk = N_ // rb

    def body(x_ref, l_ref, o_ref, acc, cnt):
        ni = pl.program_id(1)
        oh = (l_ref[...] == lax.broadcasted_iota(jnp.int32, (1, GB), 1))
        oh = oh.astype(jnp.float32)
        pa = lax.dot_general(oh, x_ref[...], (((0,), (0,)), ((), ())),
                             preferred_element_type=jnp.float32)
        pc = jnp.sum(oh, axis=0, keepdims=True)

        @pl.when(ni == 0)
        def _():
            acc[...] = pa
            cnt[...] = pc

        @pl.when(ni > 0)
        def _():
            acc[...] += pa
            cnt[...] += pc

        @pl.when(ni == nblk - 1)
        def _():
            o_ref[...] = acc[...] / jnp.maximum(cnt[...], 1.0).T

    return pl.pallas_call(
        body,
        grid=(F // fb, nblk),
        in_specs=[pl.BlockSpec((rb, fb), lambda f, n_: (n_, f)),
                  pl.BlockSpec((rb, 1), lambda f, n_: (n_, 0))],
        out_specs=pl.BlockSpec((GB, fb), lambda f, n_: (0, f)),
        out_shape=jax.ShapeDtypeStruct((GB, F), jnp.float32),
        scratch_shapes=[pltpu.VMEM((GB, fb), jnp.float32),
                        pltpu.VMEM((1, GB), jnp.float32)],
    )(x, lbi)


def _expand64(idx, table, rb):
    N_ = idx.shape[0]
    F = table.shape[1]

    def body(l_ref, t_ref, o_ref):
        oh = (l_ref[...] == lax.broadcasted_iota(jnp.int32, (1, GB), 1))
        o_ref[...] = jnp.dot(oh.astype(jnp.float32), t_ref[...],
                             preferred_element_type=jnp.float32)

    return pl.pallas_call(
        body,
        grid=(N_ // rb,),
        in_specs=[pl.BlockSpec((rb, 1), lambda i: (i, 0)),
                  pl.BlockSpec((GB, F), lambda i: (0, 0))],
        out_specs=pl.BlockSpec((rb, F), lambda i: (i, 0)),
        out_shape=jax.ShapeDtypeStruct((N_, F), jnp.float32),
    )(idx, table)


def _embed_ctx(weeks, minutes, wtab, mtab):
    R = weeks.shape[0]

    def body(w_ref, m_ref, wt_ref, mt_ref, o_ref):
        ohw = (w_ref[...] == lax.broadcasted_iota(jnp.int32, (1, 8), 1))
        ohm = (m_ref[...] == lax.broadcasted_iota(jnp.int32, (1, 288), 1))
        o_ref[...] = (
            jnp.dot(ohw.astype(jnp.float32), wt_ref[...],
                    preferred_element_type=jnp.float32)
            + jnp.dot(ohm.astype(jnp.float32), mt_ref[...],
                      preferred_element_type=jnp.float32))

    return pl.pallas_call(
        body,
        grid=(1,),
        in_specs=[pl.BlockSpec((R, 1), lambda i: (0, 0)),
                  pl.BlockSpec((R, 1), lambda i: (0, 0)),
                  pl.BlockSpec((8, 128), lambda i: (0, 0)),
                  pl.BlockSpec((288, 128), lambda i: (0, 0))],
        out_specs=pl.BlockSpec((R, 128), lambda i: (0, 0)),
        out_shape=jax.ShapeDtypeStruct((R, 128), jnp.float32),
    )(weeks, minutes, wtab, mtab)


def _region(emb_rolled, W, b, gsi):
    def body(e_ref, w_ref, b_ref, g_ref, o_ref):
        gemb = _gelu(jnp.dot(e_ref[...], w_ref[...],
                             preferred_element_type=jnp.float32) + b_ref[...])
        oh = (g_ref[...] == lax.broadcasted_iota(jnp.int32, (1, GB), 1))
        o_ref[...] = jnp.dot(oh.astype(jnp.float32), gemb,
                             preferred_element_type=jnp.float32)

    return pl.pallas_call(
        body,
        grid=(1,),
        in_specs=[pl.BlockSpec((GB, 64), lambda i: (0, 0)),
                  pl.BlockSpec((64, 64), lambda i: (0, 0)),
                  pl.BlockSpec((1, 64), lambda i: (0, 0)),
                  pl.BlockSpec((GB, 1), lambda i: (0, 0))],
        out_specs=pl.BlockSpec((GB, 64), lambda i: (0, 0)),
        out_shape=jax.ShapeDtypeStruct((GB, 64), jnp.float32),
    )(emb_rolled, W, b.reshape(1, -1), gsi)


def _ab_build(lbi, alpha_flat, beta, rb):
    N_ = lbi.shape[0]

    def body(l_ref, a_ref, b_ref, o_ref):
        oh = (l_ref[...] == lax.broadcasted_iota(jnp.int32, (1, GB), 1))
        aexp = jnp.dot(oh.astype(jnp.float32), a_ref[...],
                       preferred_element_type=jnp.float32)
        bt = b_ref[...]
        for t in range(S):
            o_ref[:, t * 256:t * 256 + 128] = aexp[:, t * 128:(t + 1) * 128]
            o_ref[:, t * 256 + 128:(t + 1) * 256] = bt

    return pl.pallas_call(
        body,
        grid=(N_ // rb,),
        in_specs=[pl.BlockSpec((rb, 1), lambda i: (i, 0)),
                  pl.BlockSpec((GB, S * 128), lambda i: (0, 0)),
                  pl.BlockSpec((rb, 128), lambda i: (i, 0))],
        out_specs=pl.BlockSpec((rb, S * 256), lambda i: (i, 0)),
        out_shape=jax.ShapeDtypeStruct((N_, S * 256), jnp.float32),
    )(lbi, alpha_flat, beta)


def _fusion(ab, s1, s2, W1, b1, W2, b2, rb):
    R, C = ab.shape
    n = float(R)

    def body(x_ref, s1_ref, s2_ref, w1_ref, b1_ref, w2_ref, b2_ref, o_ref):
        m = s1_ref[...] / n
        v = s2_ref[...] / n - m * m
        xn = (x_ref[...] - m) * lax.rsqrt(v + EPS)
        h1 = _gelu(jnp.dot(xn, w1_ref[...],
                           preferred_element_type=jnp.float32) + b1_ref[...])
        o_ref[...] = (jnp.dot(h1, w2_ref[...],
                              preferred_element_type=jnp.float32)
                      + b2_ref[...] + xn)

    return pl.pallas_call(
        body,
        grid=(R // rb,),
        in_specs=[pl.BlockSpec((rb, C), lambda i: (i, 0)),
                  pl.BlockSpec((1, C), lambda i: (0, 0)),
                  pl.BlockSpec((1, C), lambda i: (0, 0)),
                  pl.BlockSpec((C, 256), lambda i: (0, 0)),
                  pl.BlockSpec((1, 256), lambda i: (0, 0)),
                  pl.BlockSpec((256, C), lambda i: (0, 0)),
                  pl.BlockSpec((1, C), lambda i: (0, 0))],
        out_specs=pl.BlockSpec((rb, C), lambda i: (i, 0)),
        out_shape=jax.ShapeDtypeStruct((R, C), jnp.float32),
    )(ab, s1, s2, W1, b1.reshape(1, -1), W2, b2.reshape(1, -1))


def kernel(weeks, minutes, global_spatial_idx, edge_index, traffic_h,
           local_batch_idx, local_spatial_idx, local_spatial_feature, params):
    p = params
    src = edge_index[0].astype(jnp.int32)
    dst = edge_index[1].astype(jnp.int32)
    lbi = local_batch_idx.astype(jnp.int32).reshape(N, 1)

    z128 = jnp.zeros((STRIPE, 128), jnp.float32)
    ones128 = jnp.ones((K, 128), jnp.float32)

    oo, oi = _sc_degrees(src, dst, ones128, z128)
    douts = _deg_scale(oo[0, :N, :1], oo[1, :N, :1])
    dins = _deg_scale(oi[0, :N, :1], oi[1, :N, :1])
    douts_rep = jnp.repeat(douts, S, axis=0)
    dins_tile = jnp.tile(dins, (S, 1))

    x = traffic_h.reshape(N * S, 128)
    s1, s2 = _bn_stats(x, 1000)
    h_bn = _bn_apply(x, s1, s2, 1000)
    h_pre = _rowscale(h_bn, douts_rep, 1000).reshape(N, S, 128)
    h_pre_t = h_pre.transpose(1, 0, 2)

    a1 = [_sc_agg(h_pre_t[t], src, dst, z128, F=128) for t in range(S)]
    A0 = jnp.concatenate([a[0, :N] for a in a1], axis=0)
    A1 = jnp.concatenate([a[1, :N] for a in a1], axis=0)
    y1s = _mm(A0, p['enc_W0'], p['enc_b0'], 1000, x1=A1,
              pre=dins_tile, post=jnp.tile(douts, (S, 1)), act=True)
    y1r = y1s.reshape(S, N, 128)

    a2 = [_sc_agg(y1r[t], src, dst, z128, F=128) for t in range(S)]
    B0 = jnp.concatenate([a[0, :N] for a in a2], axis=0)
    B1 = jnp.concatenate([a[1, :N] for a in a2], axis=0)
    y2 = _mm(B0, p['enc_W1'], p['enc_b1'], 1000, x1=B1,
             pre=dins_tile, act=True)
    y2n = y2.reshape(S, N, 128).transpose(1, 0, 2).reshape(N, S * 128)
    hs_enc_flat = _seg_mean(y2n, lbi, 1000)
    hs_enc = hs_enc_flat.reshape(GB, S, 128)

    (ht_enc,) = _lstm(h_bn.reshape(N, S, 128),
                      p['enc_Wih'].T, p['enc_Whh'].T,
                      p['enc_bih'] + p['enc_bhh'], 128, 1000,
                      want_seq=False, want_mean=True)

    wtab = jnp.zeros((8, 128), jnp.float32).at[:7, :6].set(p['week_emb'])
    mtab = jnp.zeros((288, 128), jnp.float32).at[:, 6:70].set(p['minute_emb'])
    xctx = _embed_ctx(weeks.astype(jnp.int32).reshape(GB * S, 1),
                      minutes.astype(jnp.int32).reshape(GB * S, 1),
                      wtab, mtab).reshape(GB, S, 128)
    ctx_WihT = jnp.zeros((128, 256), jnp.float32).at[:70, :].set(p['ctx_Wih'].T)
    lth, gth = _lstm(xctx, ctx_WihT, p['ctx_Whh'].T,
                     p['ctx_bih'] + p['ctx_bhh'], 64, GB,
                     want_seq=True, want_mean=True)

    gsp = _region(jnp.roll(p['region_emb'], 1, axis=0), p['glob_W'],
                  p['glob_b'], global_spatial_idx.astype(jnp.int32).reshape(GB, 1))

    lsi = local_spatial_idx.astype(jnp.int32)
    lsi_pair = jnp.pad(lsi >> 1, (0, NP - N))
    lrows = _sc_gather(p['link_emb'].reshape(LINKS // 2, 128), lsi_pair, F=128)
    lemb = _pick_half(lrows[:N], (lsi & 1).reshape(N, 1), 1000)
    lsp_in = jnp.concatenate(
        [local_spatial_feature, lemb, jnp.zeros((N, 32), jnp.float32)], axis=1)
    lsp_pre = _rowscale(lsp_in, douts, 1000)
    la = _sc_agg(lsp_pre, src, dst, z128, F=128)
    loc_Wp = jnp.concatenate([p['loc_W'], jnp.zeros((32, 64), jnp.float32)])
    lsp = _mm(la[0, :N], loc_Wp, p['loc_b'], 1000, x1=la[1, :N],
              pre=dins, act=True)

    gth_sel = _expand64(lbi, gth, 1000)
    htd_in = jnp.concatenate([ht_enc, gth_sel, lsp], axis=1)
    t1, t2 = _bn_stats(htd_in, 1000)
    htd = _bn_apply(htd_in, t1, t2, 1000, scale=douts)
    ca = _sc_agg(htd[:, :128], src, dst, z128, F=128)
    cb = _sc_agg(htd[:, 128:], src, dst, z128, F=128)
    C0 = jnp.concatenate([ca[0, :N], cb[0, :N]], axis=1)
    C1 = jnp.concatenate([ca[1, :N], cb[1, :N]], axis=1)
    ht1s = _mm(C0, p['dec_W0'], p['dec_b0'], 1000, x1=C1,
               pre=dins, post=douts, act=True)
    da = _sc_agg(ht1s, src, dst, z128, F=128)
    beta = _mm(da[0, :N], p['dec_W1'], p['dec_b1'], 1000, x1=da[1, :N],
               pre=dins, res=ht_enc, act=True)

    c_ht = jnp.concatenate(
        [lth, jnp.broadcast_to(gsp[:, None, :], (GB, S, 64))], axis=2)
    hsd_in = jnp.concatenate([hs_enc, c_ht], axis=2).reshape(GB * S, 256)
    u1, u2 = _bn_stats(hsd_in, GB * S)
    hs_d = _bn_apply(hsd_in, u1, u2, GB * S).reshape(GB, S, 256)
    (alpha,) = _lstm(hs_d, p['dec_Wih'].T, p['dec_Whh'].T,
                     p['dec_bih'] + p['dec_bhh'], 128, GB,
                     want_seq=True, want_mean=False, res=hs_enc)

    ab = _ab_build(lbi, alpha.reshape(GB, S * 128), beta, 1000)
    ab_flat = ab.reshape(N * S, 256)
    f1, f2 = _bn_stats(ab_flat, 1000)
    out = _fusion(ab_flat, f1, f2, p['fus_W1'], p['fus_b1'],
                  p['fus_W2'], p['fus_b2'], 1000)
    return out.reshape(N, S, 256)


# speedup vs baseline: 3.6725x; 1.1590x over previous
import functools
import math

import jax
import jax.numpy as jnp
from jax import lax
from jax.experimental import pallas as pl
from jax.experimental.pallas import tpu as pltpu
from jax.experimental.pallas import tpu_sc as plsc

N = 10000
E = 320000
GB = 64
LINKS = 100000
S = 12
NTILES = 32
EPT = E // NTILES
K = 80
NCH = EPT // K
S0 = 632
SL = N - 15 * S0
KD = 200
NCHD = EPT // KD
GNP = 10240
EPS = 1e-5

_mesh = lambda: plsc.VectorSubcoreMesh(core_axis_name="c", subcore_axis_name="s")


@functools.partial(jax.jit, static_argnames=("T",))
def _sc_agg_multi(table, src2, dst2, zeros_stripe, T):
    @functools.partial(
        pl.kernel,
        out_type=jax.ShapeDtypeStruct((2 * T * N, 128), jnp.float32),
        mesh=_mesh(),
        scratch_types=[
            pltpu.VMEM((K,), jnp.int32),
            pltpu.VMEM((K,), jnp.int32),
            pltpu.VMEM((K,), jnp.int32),
            pltpu.VMEM((K,), jnp.int32),
            pltpu.VMEM((K, 128), jnp.float32),
            pltpu.VMEM((K, 128), jnp.float32),
            pltpu.VMEM_SHARED((N, 128), jnp.float32),
            pltpu.SemaphoreType.DMA,
            pltpu.SemaphoreType.DMA,
        ],
    )
    def k(tab_hbm, src_hbm, dst_hbm, z_hbm, out_hbm,
          s0, s1, d0, d1, r0, r1, acc, sem0, sem1):
        c = lax.axis_index("c")
        s = lax.axis_index("s")
        w = c * 16 + s

        def zero_acc():
            @pl.when(s < 15)
            def _():
                pltpu.sync_copy(z_hbm, acc.at[pl.ds(s * S0, S0)])

            @pl.when(s == 15)
            def _():
                pltpu.sync_copy(z_hbm.at[pl.ds(0, SL)],
                                acc.at[pl.ds(15 * S0, SL)])

        def dump_acc(obase):
            @pl.when(s < 15)
            def _():
                pltpu.sync_copy(acc.at[pl.ds(s * S0, S0)],
                                out_hbm.at[pl.ds(obase + s * S0, S0)])

            @pl.when(s == 15)
            def _():
                pltpu.sync_copy(acc.at[pl.ds(15 * S0, SL)],
                                out_hbm.at[pl.ds(obase + 15 * S0, SL)])

        def load_idx(sbuf, dbuf, soff, i):
            pltpu.sync_copy(src_hbm.at[pl.ds(soff + i * K, K)], sbuf)
            pltpu.sync_copy(dst_hbm.at[pl.ds(w * EPT + i * K, K)], dbuf)

        def gather(sbuf, rbuf, sem):
            pltpu.async_copy(tab_hbm.at[sbuf], rbuf, sem)

        def waitg(rbuf, sem):
            pltpu.make_async_copy(tab_hbm.at[s0], rbuf, sem).wait()

        def scatter(dbuf, rbuf):
            pltpu.sync_copy(rbuf, acc.at[dbuf], add=True)

        def step(t, carry):
            soff = t * E + w * EPT
            zero_acc()
            plsc.subcore_barrier()
            load_idx(s0, d0, soff, 0)
            gather(s0, r0, sem0)

            def pair(j, carry2):
                i0 = 2 * j
                i1 = i0 + 1

                @pl.when(i1 < NCH)
                def _():
                    load_idx(s1, d1, soff, i1)
                    gather(s1, r1, sem1)

                waitg(r0, sem0)
                scatter(d0, r0)

                @pl.when(i1 + 1 < NCH)
                def _():
                    load_idx(s0, d0, soff, i1 + 1)
                    gather(s0, r0, sem0)

                @pl.when(i1 < NCH)
                def _():
                    waitg(r1, sem1)
                    scatter(d1, r1)

                return carry2

            lax.fori_loop(0, (NCH + 1) // 2, pair, 0)
            plsc.subcore_barrier()
            dump_acc(c * (T * N) + t * N)
            plsc.subcore_barrier()
            return carry

        lax.fori_loop(0, T, step, 0)

    return k(table, src2, dst2, zeros_stripe)


@jax.jit
def _sc_degrees(src, dst, ones_rows, zeros_stripe):
    @functools.partial(
        pl.kernel,
        out_type=(
            jax.ShapeDtypeStruct((2 * N, 128), jnp.float32),
            jax.ShapeDtypeStruct((2 * N, 128), jnp.float32),
        ),
        mesh=_mesh(),
        scratch_types=[
            pltpu.VMEM((KD,), jnp.int32),
            pltpu.VMEM((KD, 128), jnp.float32),
            pltpu.VMEM_SHARED((N, 128), jnp.float32),
        ],
    )
    def k(src_hbm, dst_hbm, ones_hbm, z_hbm, oo_hbm, oi_hbm,
          idx_v, ones_v, acc):
        c = lax.axis_index("c")
        s = lax.axis_index("s")
        base = (c * 16 + s) * EPT

        def zero_acc():
            @pl.when(s < 15)
            def _():
                pltpu.sync_copy(z_hbm, acc.at[pl.ds(s * S0, S0)])

            @pl.when(s == 15)
            def _():
                pltpu.sync_copy(z_hbm.at[pl.ds(0, SL)],
                                acc.at[pl.ds(15 * S0, SL)])

        def dump_acc(out_hbm):
            @pl.when(s < 15)
            def _():
                pltpu.sync_copy(acc.at[pl.ds(s * S0, S0)],
                                out_hbm.at[pl.ds(c * N + s * S0, S0)])

            @pl.when(s == 15)
            def _():
                pltpu.sync_copy(acc.at[pl.ds(15 * S0, SL)],
                                out_hbm.at[pl.ds(c * N + 15 * S0, SL)])

        pltpu.sync_copy(ones_hbm, ones_v)
        zero_acc()
        plsc.subcore_barrier()

        def pass_over(idx_hbm, out_hbm, needs_rezero):
            def body(i, carry):
                pltpu.sync_copy(idx_hbm.at[pl.ds(base + i * KD, KD)], idx_v)
                pltpu.sync_copy(ones_v, acc.at[idx_v], add=True)
                return carry

            lax.fori_loop(0, NCHD, body, 0)
            plsc.subcore_barrier()
            dump_acc(out_hbm)
            if needs_rezero:
                zero_acc()
                plsc.subcore_barrier()

        pass_over(src_hbm, oo_hbm, True)
        pass_over(dst_hbm, oi_hbm, False)

    oo, oi = k(src, dst, ones_rows, zeros_stripe)
    return oo.reshape(2, N, 128), oi.reshape(2, N, 128)


@functools.partial(jax.jit, static_argnames=("F",))
def _sc_gather(table, idx, F):
    BPT = GNP // NTILES

    @functools.partial(
        pl.kernel,
        out_type=jax.ShapeDtypeStruct((GNP, F), jnp.float32),
        mesh=_mesh(),
        scratch_types=[
            pltpu.VMEM((BPT,), jnp.int32),
            pltpu.VMEM((BPT, F), jnp.float32),
            pltpu.SemaphoreType.DMA,
        ],
    )
    def k(table_hbm, idx_hbm, out_hbm, idx_v, rows_v, sem):
        c = lax.axis_index("c")
        s = lax.axis_index("s")
        base = (c * 16 + s) * BPT
        pltpu.sync_copy(idx_hbm.at[pl.ds(base, BPT)], idx_v)
        pltpu.async_copy(table_hbm.at[idx_v], rows_v, sem).wait()
        pltpu.sync_copy(rows_v, out_hbm.at[pl.ds(base, BPT)])

    return k(table, idx)


def _gelu(x):
    return 0.5 * x * (1.0 + lax.erf(x * (1.0 / math.sqrt(2.0))))


def _bn_stats(x, rb):
    R, C = x.shape

    def body(x_ref, s1_ref, s2_ref):
        i = pl.program_id(0)
        xb = x_ref[...]
        ps = jnp.sum(xb, axis=0, keepdims=True)
        pss = jnp.sum(xb * xb, axis=0, keepdims=True)

        @pl.when(i == 0)
        def _():
            s1_ref[...] = ps
            s2_ref[...] = pss

        @pl.when(i > 0)
        def _():
            s1_ref[...] += ps
            s2_ref[...] += pss

    return pl.pallas_call(
        body,
        grid=(R // rb,),
        in_specs=[pl.BlockSpec((rb, C), lambda i: (i, 0))],
        out_specs=(pl.BlockSpec((1, C), lambda i: (0, 0)),
                   pl.BlockSpec((1, C), lambda i: (0, 0))),
        out_shape=(jax.ShapeDtypeStruct((1, C), jnp.float32),
                   jax.ShapeDtypeStruct((1, C), jnp.float32)),
    )(x)


def _bn_stats_fold(x, rb, H):
    R, C = x.shape

    def body(x_ref, s1_ref, s2_ref):
        i = pl.program_id(0)
        xb = x_ref[...]
        ps = jnp.zeros((1, 128), jnp.float32)
        pss = jnp.zeros((1, 128), jnp.float32)
        for h in range(H):
            blk = xb[:, h * 128:(h + 1) * 128]
            ps = ps + jnp.sum(blk, axis=0, keepdims=True)
            pss = pss + jnp.sum(blk * blk, axis=0, keepdims=True)

        @pl.when(i == 0)
        def _():
            s1_ref[...] = ps
            s2_ref[...] = pss

        @pl.when(i > 0)
        def _():
            s1_ref[...] += ps
            s2_ref[...] += pss

    return pl.pallas_call(
        body,
        grid=(R // rb,),
        in_specs=[pl.BlockSpec((rb, C), lambda i: (i, 0))],
        out_specs=(pl.BlockSpec((1, 128), lambda i: (0, 0)),
                   pl.BlockSpec((1, 128), lambda i: (0, 0))),
        out_shape=(jax.ShapeDtypeStruct((1, 128), jnp.float32),
                   jax.ShapeDtypeStruct((1, 128), jnp.float32)),
    )(x)


def _bn_apply_T(x, s1, s2, rb, H, n_rows, scale=None,
                want_plain=True, want_scaled=False):
    R, C = x.shape
    nb = R // rb
    with_scale = scale is not None

    def body(*refs):
        it = iter(refs)
        x_ref = next(it)
        s1_ref = next(it)
        s2_ref = next(it)
        sc_ref = next(it) if with_scale else None
        outs = [next(it) for _ in range(int(want_plain) + int(want_scaled))]
        m = s1_ref[...] / n_rows
        v = s2_ref[...] / n_rows - m * m
        y = (x_ref[...] - m) * lax.rsqrt(v + EPS)
        oi = 0
        if want_plain:
            outs[oi][...] = y
            oi += 1
        if want_scaled:
            outs[oi][...] = y * sc_ref[...]

    in_specs = [pl.BlockSpec((rb, 128), lambda i, h: (i, h)),
                pl.BlockSpec((1, 128), lambda i, h: (0, h)),
                pl.BlockSpec((1, 128), lambda i, h: (0, h))]
    args = [x, s1, s2]
    if with_scale:
        in_specs.append(pl.BlockSpec((rb, 1), lambda i, h: (i, 0)))
        args.append(scale)
    ospec = pl.BlockSpec((rb, 128), lambda i, h: (h * nb + i, 0))
    oshape = jax.ShapeDtypeStruct((H * R, 128), jnp.float32)
    nout = int(want_plain) + int(want_scaled)
    outs = pl.pallas_call(
        body,
        grid=(nb, H),
        in_specs=in_specs,
        out_specs=tuple([ospec] * nout),
        out_shape=tuple([oshape] * nout),
    )(*args)
    return outs


def _mm(x0, W, b, rb, rows=None, x1=None, x1_boff=0, pre=None, post=None,
        res=None, act=True):
    R = rows if rows is not None else x0.shape[0]
    Kd = x0.shape[1]
    M = W.shape[1]
    f_x1, f_pre, f_post, f_res = (x1 is not None, pre is not None,
                                  post is not None, res is not None)

    def body(*refs):
        it = iter(refs)
        x_ref = next(it)
        x1_ref = next(it) if f_x1 else None
        W_ref = next(it)
        b_ref = next(it)
        pre_ref = next(it) if f_pre else None
        post_ref = next(it) if f_post else None
        res_ref = next(it) if f_res else None
        o_ref = next(it)
        x = x_ref[...]
        if f_x1:
            x = x + x1_ref[...]
        if f_pre:
            x = x * pre_ref[...]
        y = jnp.dot(x, W_ref[...], preferred_element_type=jnp.float32) + b_ref[...]
        if act:
            y = _gelu(y)
        if f_post:
            y = y * post_ref[...]
        if f_res:
            y = y + res_ref[...]
        o_ref[...] = y

    in_specs = [pl.BlockSpec((rb, Kd), lambda i: (i, 0))]
    args = [x0]
    if f_x1:
        off = x1_boff
        in_specs.append(pl.BlockSpec((rb, Kd), lambda i: (i + off, 0)))
        args.append(x1)
    in_specs += [pl.BlockSpec((Kd, M), lambda i: (0, 0)),
                 pl.BlockSpec((1, M), lambda i: (0, 0))]
    args += [W, b.reshape(1, M)]
    if f_pre:
        in_specs.append(pl.BlockSpec((rb, 1), lambda i: (i, 0)))
        args.append(pre)
    if f_post:
        in_specs.append(pl.BlockSpec((rb, 1), lambda i: (i, 0)))
        args.append(post)
    if f_res:
        in_specs.append(pl.BlockSpec((rb, M), lambda i: (i, 0)))
        args.append(res)
    return pl.pallas_call(
        body,
        grid=(R // rb,),
        in_specs=in_specs,
        out_specs=pl.BlockSpec((rb, M), lambda i: (i, 0)),
        out_shape=jax.ShapeDtypeStruct((R, M), jnp.float32),
    )(*args)


def _mm_k256(agg, W, b, rb, pre, post):
    nb = N // rb

    def body(x0a_ref, x0b_ref, x1a_ref, x1b_ref, w_ref, b_ref,
             pre_ref, post_ref, o_ref):
        t0 = (x0a_ref[...] + x0b_ref[...]) * pre_ref[...]
        t1 = (x1a_ref[...] + x1b_ref[...]) * pre_ref[...]
        w = w_ref[...]
        y = (jnp.dot(t0, w[0:128, :], preferred_element_type=jnp.float32)
             + jnp.dot(t1, w[128:256, :], preferred_element_type=jnp.float32)
             + b_ref[...])
        o_ref[...] = _gelu(y) * post_ref[...]

    return pl.pallas_call(
        body,
        grid=(nb,),
        in_specs=[pl.BlockSpec((rb, 128), lambda i: (i, 0)),
                  pl.BlockSpec((rb, 128), lambda i: (i + 2 * nb, 0)),
                  pl.BlockSpec((rb, 128), lambda i: (i + nb, 0)),
                  pl.BlockSpec((rb, 128), lambda i: (i + 3 * nb, 0)),
                  pl.BlockSpec((256, 128), lambda i: (0, 0)),
                  pl.BlockSpec((1, 128), lambda i: (0, 0)),
                  pl.BlockSpec((rb, 1), lambda i: (i, 0)),
                  pl.BlockSpec((rb, 1), lambda i: (i, 0))],
        out_specs=pl.BlockSpec((rb, 128), lambda i: (i, 0)),
        out_shape=jax.ShapeDtypeStruct((N, 128), jnp.float32),
    )(agg, agg, agg, agg, W, b.reshape(1, -1), pre, post)


def _pick_half(rows, parity, rb):
    R = rows.shape[0]

    def body(r_ref, p_ref, o_ref):
        o_ref[...] = jnp.where(p_ref[...] > 0, r_ref[:, 64:128], r_ref[:, 0:64])

    return pl.pallas_call(
        body,
        grid=(R // rb,),
        in_specs=[pl.BlockSpec((rb, 128), lambda i: (i, 0)),
                  pl.BlockSpec((rb, 1), lambda i: (i, 0))],
        out_specs=pl.BlockSpec((rb, 64), lambda i: (i, 0)),
        out_shape=jax.ShapeDtypeStruct((R, 64), jnp.float32),
    )(rows, parity)


def _deg_scale(a0, a1):
    R = a0.shape[0]
    rb = 1000

    def body(a_ref, b_ref, o_ref):
        o_ref[...] = lax.rsqrt(jnp.maximum(a_ref[...] + b_ref[...], 1.0))

    return pl.pallas_call(
        body,
        grid=(R // rb,),
        in_specs=[pl.BlockSpec((rb, 1), lambda i: (i, 0)),
                  pl.BlockSpec((rb, 1), lambda i: (i, 0))],
        out_specs=pl.BlockSpec((rb, 1), lambda i: (i, 0)),
        out_shape=jax.ShapeDtypeStruct((R, 1), jnp.float32),
    )(a0, a1)


def _rowscale(x, sc, rb):
    R, C = x.shape

    def body(x_ref, s_ref, o_ref):
        o_ref[...] = x_ref[...] * s_ref[...]

    return pl.pallas_call(
        body,
        grid=(R // rb,),
        in_specs=[pl.BlockSpec((rb, C), lambda i: (i, 0)),
                  pl.BlockSpec((rb, 1), lambda i: (i, 0))],
        out_specs=pl.BlockSpec((rb, C), lambda i: (i, 0)),
        out_shape=jax.ShapeDtypeStruct((R, C), jnp.float32),
    )(x, sc)


def _lstm(x, WihT, WhhT, bias, H, rb, want_seq, want_mean, res=None,
          layout="ns"):
    if layout == "ns":
        B, S_, I = x.shape
    else:
        S_, B, I = x.shape
    f_res = res is not None

    def body(*refs):
        it = iter(refs)
        x_ref = next(it)
        wi_ref = next(it)
        wh_ref = next(it)
        b_ref = next(it)
        res_ref = next(it) if f_res else None
        outs = [next(it) for _ in range(int(want_seq) + int(want_mean))]
        h = jnp.zeros((rb, H), jnp.float32)
        c = jnp.zeros((rb, H), jnp.float32)
        acc = jnp.zeros((rb, H), jnp.float32)
        wi = wi_ref[...]
        wh = wh_ref[...]
        bb = b_ref[...]
        for t in range(S_):
            xt = x_ref[:, t, :] if layout == "ns" else x_ref[t]
            g = (jnp.dot(xt, wi, preferred_element_type=jnp.float32)
                 + jnp.dot(h, wh, preferred_element_type=jnp.float32) + bb)
            i_g = jax.nn.sigmoid(g[:, 0:H])
            f_g = jax.nn.sigmoid(g[:, H:2 * H])
            g_g = jnp.tanh(g[:, 2 * H:3 * H])
            o_g = jax.nn.sigmoid(g[:, 3 * H:4 * H])
            c = f_g * c + i_g * g_g
            h = o_g * jnp.tanh(c)
            oi = 0
            if want_seq:
                y = h
                if f_res:
                    y = y + res_ref[:, t, :]
                outs[oi][:, t, :] = y
                oi += 1
            if want_mean:
                acc = acc + h
        if want_mean:
            outs[-1][...] = acc * (1.0 / S_)

    if layout == "ns":
        xspec = pl.BlockSpec((rb, S_, I), lambda i: (i, 0, 0))
    else:
        xspec = pl.BlockSpec((S_, rb, I), lambda i: (0, i, 0))
    in_specs = [xspec,
                pl.BlockSpec((I, 4 * H), lambda i: (0, 0)),
                pl.BlockSpec((H, 4 * H), lambda i: (0, 0)),
                pl.BlockSpec((1, 4 * H), lambda i: (0, 0))]
    args = [x, WihT, WhhT, bias.reshape(1, -1)]
    if f_res:
        in_specs.append(pl.BlockSpec((rb, S_, H), lambda i: (i, 0, 0)))
        args.append(res)
    out_specs, out_shape = [], []
    if want_seq:
        out_specs.append(pl.BlockSpec((rb, S_, H), lambda i: (i, 0, 0)))
        out_shape.append(jax.ShapeDtypeStruct((B, S_, H), jnp.float32))
    if want_mean:
        out_specs.append(pl.BlockSpec((rb, H), lambda i: (i, 0)))
        out_shape.append(jax.ShapeDtypeStruct((B, H), jnp.float32))
    outs = pl.pallas_call(
        body,
        grid=(B // rb,),
        in_specs=in_specs,
        out_specs=tuple(out_specs),
        out_shape=tuple(out_shape),
    )(*args)
    return outs


def _seg_mean_T(x, lbi, rb):
    nblk = N // rb

    def body(x_ref, l_ref, o_ref, acc, cnt):
        ni = pl.program_id(1)
        oh = (l_ref[...] == lax.broadcasted_iota(jnp.int32, (1, GB), 1))
        oh = oh.astype(jnp.float32)
        pa = lax.dot_general(oh, x_ref[...], (((0,), (0,)), ((), ())),
                             preferred_element_type=jnp.float32)
        pc = jnp.sum(oh, axis=0, keepdims=True)

        @pl.when(ni == 0)
        def _():
            acc[...] = pa
            cnt[...] = pc

        @pl.when(ni > 0)
        def _():
            acc[...] += pa
            cnt[...] += pc

        @pl.when(ni == nblk - 1)
        def _():
            o_ref[...] = acc[...] / jnp.maximum(cnt[...], 1.0).T

    return pl.pallas_call(
        body,
        grid=(S, nblk),
        in_specs=[pl.BlockSpec((rb, 128), lambda t, n_: (t * nblk + n_, 0)),
                  pl.BlockSpec((rb, 1), lambda t, n_: (n_, 0))],
        out_specs=pl.BlockSpec((GB, 128), lambda t, n_: (t, 0)),
        out_shape=jax.ShapeDtypeStruct((S * GB, 128), jnp.float32),
        scratch_shapes=[pltpu.VMEM((GB, 128), jnp.float32),
                        pltpu.VMEM((1, GB), jnp.float32)],
    )(x, lbi)


def _expand64(idx, table, rb):
    N_ = idx.shape[0]
    F = table.shape[1]

    def body(l_ref, t_ref, o_ref):
        oh = (l_ref[...] == lax.broadcasted_iota(jnp.int32, (1, GB), 1))
        o_ref[...] = jnp.dot(oh.astype(jnp.float32), t_ref[...],
                             preferred_element_type=jnp.float32)

    return pl.pallas_call(
        body,
        grid=(N_ // rb,),
        in_specs=[pl.BlockSpec((rb, 1), lambda i: (i, 0)),
                  pl.BlockSpec((GB, F), lambda i: (0, 0))],
        out_specs=pl.BlockSpec((rb, F), lambda i: (i, 0)),
        out_shape=jax.ShapeDtypeStruct((N_, F), jnp.float32),
    )(idx, table)


def _embed_ctx(weeks, minutes, wtab, mtab):
    R = weeks.shape[0]

    def body(w_ref, m_ref, wt_ref, mt_ref, o_ref):
        ohw = (w_ref[...] == lax.broadcasted_iota(jnp.int32, (1, 8), 1))
        ohm = (m_ref[...] == lax.broadcasted_iota(jnp.int32, (1, 288), 1))
        o_ref[...] = (
            jnp.dot(ohw.astype(jnp.float32), wt_ref[...],
                    preferred_element_type=jnp.float32)
            + jnp.dot(ohm.astype(jnp.float32), mt_ref[...],
                      preferred_element_type=jnp.float32))

    return pl.pallas_call(
        body,
        grid=(1,),
        in_specs=[pl.BlockSpec((R, 1), lambda i: (0, 0)),
                  pl.BlockSpec((R, 1), lambda i: (0, 0)),
                  pl.BlockSpec((8, 128), lambda i: (0, 0)),
                  pl.BlockSpec((288, 128), lambda i: (0, 0))],
        out_specs=pl.BlockSpec((R, 128), lambda i: (0, 0)),
        out_shape=jax.ShapeDtypeStruct((R, 128), jnp.float32),
    )(weeks, minutes, wtab, mtab)


def _region(emb_rolled, W, b, gsi):
    def body(e_ref, w_ref, b_ref, g_ref, o_ref):
        gemb = _gelu(jnp.dot(e_ref[...], w_ref[...],
                             preferred_element_type=jnp.float32) + b_ref[...])
        oh = (g_ref[...] == lax.broadcasted_iota(jnp.int32, (1, GB), 1))
        o_ref[...] = jnp.dot(oh.astype(jnp.float32), gemb,
                             preferred_element_type=jnp.float32)

    return pl.pallas_call(
        body,
        grid=(1,),
        in_specs=[pl.BlockSpec((GB, 64), lambda i: (0, 0)),
                  pl.BlockSpec((64, 64), lambda i: (0, 0)),
                  pl.BlockSpec((1, 64), lambda i: (0, 0)),
                  pl.BlockSpec((GB, 1), lambda i: (0, 0))],
        out_specs=pl.BlockSpec((GB, 64), lambda i: (0, 0)),
        out_shape=jax.ShapeDtypeStruct((GB, 64), jnp.float32),
    )(emb_rolled, W, b.reshape(1, -1), gsi)


def _ab_build(lbi, alpha_flat, beta, rb):
    N_ = lbi.shape[0]

    def body(l_ref, a_ref, b_ref, o_ref):
        oh = (l_ref[...] == lax.broadcasted_iota(jnp.int32, (1, GB), 1))
        aexp = jnp.dot(oh.astype(jnp.float32), a_ref[...],
                       preferred_element_type=jnp.float32)
        bt = b_ref[...]
        for t in range(S):
            o_ref[:, t * 256:t * 256 + 128] = aexp[:, t * 128:(t + 1) * 128]
            o_ref[:, t * 256 + 128:(t + 1) * 256] = bt

    return pl.pallas_call(
        body,
        grid=(N_ // rb,),
        in_specs=[pl.BlockSpec((rb, 1), lambda i: (i, 0)),
                  pl.BlockSpec((GB, S * 128), lambda i: (0, 0)),
                  pl.BlockSpec((rb, 128), lambda i: (i, 0))],
        out_specs=pl.BlockSpec((rb, S * 256), lambda i: (i, 0)),
        out_shape=jax.ShapeDtypeStruct((N_, S * 256), jnp.float32),
    )(lbi, alpha_flat, beta)


def _fusion(ab, s1, s2, W1, b1, W2, b2, rb):
    R, C = ab.shape
    n = float(R)

    def body(x_ref, s1_ref, s2_ref, w1_ref, b1_ref, w2_ref, b2_ref, o_ref):
        m = s1_ref[...] / n
        v = s2_ref[...] / n - m * m
        xn = (x_ref[...] - m) * lax.rsqrt(v + EPS)
        h1 = _gelu(jnp.dot(xn, w1_ref[...],
                           preferred_element_type=jnp.float32) + b1_ref[...])
        o_ref[...] = (jnp.dot(h1, w2_ref[...],
                              preferred_element_type=jnp.float32)
                      + b2_ref[...] + xn)

    return pl.pallas_call(
        body,
        grid=(R // rb,),
        in_specs=[pl.BlockSpec((rb, C), lambda i: (i, 0)),
                  pl.BlockSpec((1, C), lambda i: (0, 0)),
                  pl.BlockSpec((1, C), lambda i: (0, 0)),
                  pl.BlockSpec((C, 256), lambda i: (0, 0)),
                  pl.BlockSpec((1, 256), lambda i: (0, 0)),
                  pl.BlockSpec((256, C), lambda i: (0, 0)),
                  pl.BlockSpec((1, C), lambda i: (0, 0))],
        out_specs=pl.BlockSpec((rb, C), lambda i: (i, 0)),
        out_shape=jax.ShapeDtypeStruct((R, C), jnp.float32),
    )(ab, s1, s2, W1, b1.reshape(1, -1), W2, b2.reshape(1, -1))


def kernel(weeks, minutes, global_spatial_idx, edge_index, traffic_h,
           local_batch_idx, local_spatial_idx, local_spatial_feature, params):
    p = params
    src = edge_index[0].astype(jnp.int32)
    dst = edge_index[1].astype(jnp.int32)
    lbi = local_batch_idx.astype(jnp.int32).reshape(N, 1)

    z128 = jnp.zeros((S0, 128), jnp.float32)
    ones128 = jnp.ones((KD, 128), jnp.float32)
    dst2 = dst
    src1 = src
    tshift = (jnp.arange(S, dtype=jnp.int32) * N)[:, None]
    srcS = (src[None, :] + tshift).reshape(S * E)
    src2h = (src[None, :] + tshift[:2]).reshape(2 * E)

    oo, oi = _sc_degrees(src, dst, ones128, z128)
    douts = _deg_scale(oo[0, :, :1], oo[1, :, :1])
    dins = _deg_scale(oi[0, :, :1], oi[1, :, :1])
    douts_tile = jnp.tile(douts, (S, 1))
    dins_tile = jnp.tile(dins, (S, 1))

    x2 = traffic_h.reshape(N, S * 128)
    s1, s2 = _bn_stats_fold(x2, 1000, S)
    h_bn_T, h_pre_T = _bn_apply_T(
        x2, jnp.tile(s1, (1, S)), jnp.tile(s2, (1, S)), 1000, S,
        float(N * S), scale=douts, want_plain=True, want_scaled=True)

    g1 = _sc_agg_multi(h_pre_T, srcS, dst2, z128, T=S)
    y1s = _mm(g1, p['enc_W0'], p['enc_b0'], 1000, rows=S * N,
              x1=g1, x1_boff=(S * N) // 1000,
              pre=dins_tile, post=douts_tile, act=True)
    g2 = _sc_agg_multi(y1s, srcS, dst2, z128, T=S)
    y2 = _mm(g2, p['enc_W1'], p['enc_b1'], 1000, rows=S * N,
             x1=g2, x1_boff=(S * N) // 1000, pre=dins_tile, act=True)
    hs_enc = _seg_mean_T(y2, lbi, 1000).reshape(S, GB, 128).transpose(1, 0, 2)

    (ht_enc,) = _lstm(h_bn_T.reshape(S, N, 128),
                      p['enc_Wih'].T, p['enc_Whh'].T,
                      p['enc_bih'] + p['enc_bhh'], 128, 1000,
                      want_seq=False, want_mean=True, layout="sn")

    wtab = jnp.zeros((8, 128), jnp.float32).at[:7, :6].set(p['week_emb'])
    mtab = jnp.zeros((288, 128), jnp.float32).at[:, 6:70].set(p['minute_emb'])
    xctx = _embed_ctx(weeks.astype(jnp.int32).reshape(GB * S, 1),
                      minutes.astype(jnp.int32).reshape(GB * S, 1),
                      wtab, mtab).reshape(GB, S, 128)
    ctx_WihT = jnp.zeros((128, 256), jnp.float32).at[:70, :].set(p['ctx_Wih'].T)
    lth, gth = _lstm(xctx, ctx_WihT, p['ctx_Whh'].T,
                     p['ctx_bih'] + p['ctx_bhh'], 64, GB,
                     want_seq=True, want_mean=True)

    gsp = _region(jnp.roll(p['region_emb'], 1, axis=0), p['glob_W'],
                  p['glob_b'], global_spatial_idx.astype(jnp.int32).reshape(GB, 1))

    lsi = local_spatial_idx.astype(jnp.int32)
    lsi_pair = jnp.pad(lsi >> 1, (0, GNP - N))
    lrows = _sc_gather(p['link_emb'].reshape(LINKS // 2, 128), lsi_pair, F=128)
    lemb = _pick_half(lrows[:N], (lsi & 1).reshape(N, 1), 1000)
    lsp_in = jnp.concatenate(
        [local_spatial_feature, lemb, jnp.zeros((N, 32), jnp.float32)], axis=1)
    lsp_pre = _rowscale(lsp_in, douts, 1000)
    la = _sc_agg_multi(lsp_pre, src1, dst2, z128, T=1)
    loc_Wp = jnp.concatenate([p['loc_W'], jnp.zeros((32, 64), jnp.float32)])
    lsp = _mm(la, loc_Wp, p['loc_b'], 1000, rows=N,
              x1=la, x1_boff=N // 1000, pre=dins, act=True)

    gth_sel = _expand64(lbi, gth, 1000)
    htd_in = jnp.concatenate([ht_enc, gth_sel, lsp], axis=1)
    t1, t2 = _bn_stats(htd_in, 1000)
    (htd_T,) = _bn_apply_T(htd_in, t1, t2, 1000, 2, float(N), scale=douts,
                           want_plain=False, want_scaled=True)
    gd = _sc_agg_multi(htd_T, src2h, dst2, z128, T=2)
    ht1s = _mm_k256(gd, p['dec_W0'], p['dec_b0'], 1000, dins, douts)
    ge = _sc_agg_multi(ht1s, src1, dst2, z128, T=1)
    beta = _mm(ge, p['dec_W1'], p['dec_b1'], 1000, rows=N,
               x1=ge, x1_boff=N // 1000, pre=dins, res=ht_enc, act=True)

    c_ht = jnp.concatenate(
        [lth, jnp.broadcast_to(gsp[:, None, :], (GB, S, 64))], axis=2)
    hsd_in = jnp.concatenate([hs_enc, c_ht], axis=2).reshape(GB * S, 256)
    u1, u2 = _bn_stats(hsd_in, GB * S)
    hs_d = _bn_apply(hsd_in, u1, u2).reshape(GB, S, 256)
    (alpha,) = _lstm(hs_d, p['dec_Wih'].T, p['dec_Whh'].T,
                     p['dec_bih'] + p['dec_bhh'], 128, GB,
                     want_seq=True, want_mean=False, res=hs_enc)

    ab = _ab_build(lbi, alpha.reshape(GB, S * 128), beta, 1000)
    ab_flat = ab.reshape(N * S, 256)
    f1, f2 = _bn_stats(ab_flat, 1000)
    out = _fusion(ab_flat, f1, f2, p['fus_W1'], p['fus_b1'],
                  p['fus_W2'], p['fus_b2'], 1000)
    return out.reshape(N, S, 256)


def _bn_apply(x, s1, s2):
    R, C = x.shape
    n = float(R)

    def body(x_ref, s1_ref, s2_ref, o_ref):
        m = s1_ref[...] / n
        v = s2_ref[...] / n - m * m
        o_ref[...] = (x_ref[...] - m) * lax.rsqrt(v + EPS)

    return pl.pallas_call(
        body,
        grid=(1,),
        in_specs=[pl.BlockSpec((R, C), lambda i: (0, 0)),
                  pl.BlockSpec((1, C), lambda i: (0, 0)),
                  pl.BlockSpec((1, C), lambda i: (0, 0))],
        out_specs=pl.BlockSpec((R, C), lambda i: (0, 0)),
        out_shape=jax.ShapeDtypeStruct((R, C), jnp.float32),
    )(x, s1, s2)
